# baseline (speedup 1.0000x reference)
import functools

import jax
import jax.numpy as jnp
from jax import lax
from jax.experimental import pallas as pl
from jax.experimental.pallas import tpu as pltpu
from jax.experimental.pallas import tpu_sc as plsc

VOCAB = 1000
NUM_CORES = 2
NUM_SUBCORES = 16
NUM_WORKERS = NUM_CORES * NUM_SUBCORES
CHUNK = 40


def kernel(idx, embd_weight):
    B, T = idx.shape
    n = B * T
    per_w = n // NUM_WORKERS
    nchunks = per_w // CHUNK
    flat_idx = idx.reshape(n).astype(jnp.int32)

    mesh = plsc.VectorSubcoreMesh(core_axis_name="c", subcore_axis_name="s")

    @functools.partial(
        pl.kernel,
        out_type=jax.ShapeDtypeStruct((n, VOCAB), jnp.float32),
        mesh=mesh,
        compiler_params=pltpu.CompilerParams(use_tc_tiling_on_sc=False),
        scratch_types=[
            pltpu.VMEM((per_w,), jnp.int32),
            pltpu.VMEM((CHUNK, VOCAB), jnp.float32),
            pltpu.SemaphoreType.DMA,
            pltpu.SemaphoreType.DMA,
        ],
    )
    def gather_kernel(table_hbm, idx_hbm, out_hbm, idx_v, rows_v, gsem, ssem):
        wid = lax.axis_index("s") * NUM_CORES + lax.axis_index("c")
        base = wid * per_w
        pltpu.sync_copy(idx_hbm.at[pl.ds(base, per_w)], idx_v)

        @pl.loop(0, nchunks)
        def _(j):
            off = j * CHUNK
            pltpu.async_copy(
                table_hbm.at[idx_v.at[pl.ds(off, CHUNK)]], rows_v, gsem
            ).wait()
            pltpu.async_copy(
                rows_v, out_hbm.at[pl.ds(base + off, CHUNK)], ssem
            ).wait()

    out = gather_kernel(embd_weight, flat_idx)
    return out.reshape(B, T, VOCAB)


# speedup vs baseline: 1.0286x; 1.0286x over previous
import functools

import jax
import jax.numpy as jnp
from jax import lax
from jax.experimental import pallas as pl
from jax.experimental.pallas import tpu as pltpu
from jax.experimental.pallas import tpu_sc as plsc

VOCAB = 1000
NUM_CORES = 2
NUM_SUBCORES = 16
NUM_WORKERS = NUM_CORES * NUM_SUBCORES
CHUNK = 64


def kernel(idx, embd_weight):
    B, T = idx.shape
    n = B * T
    per_w = n // NUM_WORKERS
    nchunks = per_w // CHUNK
    flat_idx = idx.reshape(n).astype(jnp.int32)

    mesh = plsc.VectorSubcoreMesh(core_axis_name="c", subcore_axis_name="s")

    @functools.partial(
        pl.kernel,
        out_type=jax.ShapeDtypeStruct((n, VOCAB), jnp.float32),
        mesh=mesh,
        compiler_params=pltpu.CompilerParams(use_tc_tiling_on_sc=False),
        scratch_types=[
            pltpu.VMEM((per_w,), jnp.int32),
            pltpu.VMEM((CHUNK, VOCAB), jnp.float32),
            pltpu.VMEM((CHUNK, VOCAB), jnp.float32),
            pltpu.SemaphoreType.DMA,
            pltpu.SemaphoreType.DMA,
            pltpu.SemaphoreType.DMA,
            pltpu.SemaphoreType.DMA,
        ],
    )
    def gather_kernel(table_hbm, idx_hbm, out_hbm, idx_v, rows_a, rows_b,
                      gsem_a, gsem_b, wsem_a, wsem_b):
        rows = (rows_a, rows_b)
        gsem = (gsem_a, gsem_b)
        wsem = (wsem_a, wsem_b)

        wid = lax.axis_index("s") * NUM_CORES + lax.axis_index("c")
        base = wid * per_w
        pltpu.sync_copy(idx_hbm.at[pl.ds(base, per_w)], idx_v)

        def issue_gather(c, b):
            pltpu.async_copy(
                table_hbm.at[idx_v.at[pl.ds(c * CHUNK, CHUNK)]],
                rows[b], gsem[b])

        def wait_gather(b):
            pltpu.make_async_copy(
                table_hbm.at[pl.ds(0, CHUNK)], rows[b], gsem[b]).wait()

        def issue_write(c, b):
            pltpu.async_copy(
                rows[b], out_hbm.at[pl.ds(base + c * CHUNK, CHUNK)], wsem[b])

        def wait_write(b):
            pltpu.make_async_copy(
                rows[b], out_hbm.at[pl.ds(0, CHUNK)], wsem[b]).wait()

        issue_gather(0, 0)

        @pl.loop(0, nchunks - 1, step=2)
        def _(g):
            for b in (0, 1):
                c = g + b
                wait_gather(b)
                issue_gather(c + 1, 1 - b)
                issue_write(c, b)
                wait_write(b)

        last = nchunks - 1
        bl = last % 2
        wait_gather(bl)
        issue_write(last, bl)
        wait_write(bl)

    out = gather_kernel(embd_weight, flat_idx)
    return out.reshape(B, T, VOCAB)


# speedup vs baseline: 1.4160x; 1.3767x over previous
import functools

import jax
import jax.numpy as jnp
from jax import lax
from jax.experimental import pallas as pl
from jax.experimental.pallas import tpu as pltpu
from jax.experimental.pallas import tpu_sc as plsc

VOCAB = 1000
VOCAB_PAD = 1024
NUM_CORES = 2
NUM_SUBCORES = 16
NUM_WORKERS = NUM_CORES * NUM_SUBCORES
CHUNK = 40


def kernel(idx, embd_weight):
    B, T = idx.shape
    n = B * T
    per_w = n // NUM_WORKERS
    nchunks = per_w // CHUNK
    flat_idx = idx.reshape(n).astype(jnp.int32)
    table_pad = jnp.pad(embd_weight, ((0, 0), (0, VOCAB_PAD - VOCAB)))

    mesh = plsc.VectorSubcoreMesh(core_axis_name="c", subcore_axis_name="s")

    @functools.partial(
        pl.kernel,
        out_type=jax.ShapeDtypeStruct((n, VOCAB_PAD), jnp.float32),
        mesh=mesh,
        scratch_types=[
            pltpu.VMEM((per_w,), jnp.int32),
            pltpu.VMEM((CHUNK, VOCAB_PAD), jnp.float32),
            pltpu.VMEM((CHUNK, VOCAB_PAD), jnp.float32),
            pltpu.SemaphoreType.DMA,
            pltpu.SemaphoreType.DMA,
            pltpu.SemaphoreType.DMA,
            pltpu.SemaphoreType.DMA,
        ],
    )
    def gather_kernel(table_hbm, idx_hbm, out_hbm, idx_v, rows_a, rows_b,
                      gsem_a, gsem_b, wsem_a, wsem_b):
        rows = (rows_a, rows_b)
        gsem = (gsem_a, gsem_b)
        wsem = (wsem_a, wsem_b)

        wid = lax.axis_index("s") * NUM_CORES + lax.axis_index("c")
        base = wid * per_w
        pltpu.sync_copy(idx_hbm.at[pl.ds(base, per_w)], idx_v)

        def issue_gather(c, b):
            pltpu.async_copy(
                table_hbm.at[idx_v.at[pl.ds(c * CHUNK, CHUNK)]],
                rows[b], gsem[b])

        def wait_gather(b):
            pltpu.make_async_copy(
                table_hbm.at[pl.ds(0, CHUNK)], rows[b], gsem[b]).wait()

        def issue_write(c, b):
            pltpu.async_copy(
                rows[b], out_hbm.at[pl.ds(base + c * CHUNK, CHUNK)], wsem[b])

        def wait_write(b):
            pltpu.make_async_copy(
                rows[b], out_hbm.at[pl.ds(0, CHUNK)], wsem[b]).wait()

        issue_gather(0, 0)

        @pl.loop(0, nchunks - 2, step=2)
        def _(g):
            for b in (0, 1):
                c = g + b
                wait_gather(b)
                issue_gather(c + 1, 1 - b)
                issue_write(c, b)
                wait_write(b)

        c1 = nchunks - 2
        b1 = c1 % 2
        wait_gather(b1)
        issue_gather(nchunks - 1, 1 - b1)
        issue_write(c1, b1)
        wait_write(b1)

        c2 = nchunks - 1
        b2 = c2 % 2
        wait_gather(b2)
        issue_write(c2, b2)
        wait_write(b2)

    out = gather_kernel(table_pad, flat_idx)
    return out[:, :VOCAB].reshape(B, T, VOCAB)
